# baseline (speedup 1.0000x reference)
import functools

import jax
import jax.numpy as jnp
from jax import lax
from jax.experimental import pallas as pl
from jax.experimental.pallas import tpu as pltpu
from jax.experimental.pallas import tpu_sc as plsc

HIDDEN = 1024
TOTAL_ROWS = 4 * 8192
NUM_WORKERS = 32
ROWS_PER_WORKER = TOTAL_ROWS // NUM_WORKERS
NLISTS = 3
B = 32
NB_MAX = ROWS_PER_WORKER // B + 1

_mesh = plsc.VectorSubcoreMesh(core_axis_name="c", subcore_axis_name="s")


@functools.partial(
    pl.kernel,
    mesh=_mesh,
    out_type=jax.ShapeDtypeStruct((TOTAL_ROWS, HIDDEN), jnp.float32),
    compiler_params=pltpu.CompilerParams(needs_layout_passes=False),
    scratch_types=[
        pltpu.VMEM((ROWS_PER_WORKER,), jnp.int32),
        pltpu.VMEM((NB_MAX, B), jnp.int32),
        pltpu.VMEM((NB_MAX, B), jnp.int32),
        pltpu.VMEM((NB_MAX, B), jnp.int32),
        pltpu.VMEM((32,), jnp.int32),
        pltpu.VMEM((NLISTS * B, HIDDEN), jnp.float32),
        pltpu.SemaphoreType.DMA,
        pltpu.SemaphoreType.DMA,
    ],
)
def _scatter_kernel(idx_hbm, table_hbm, out_hbm, idx_v, pos0_v, pos1_v,
                    pos2_v, tidx_v, rep_v, gsem, sem):
    pos_refs = (pos0_v, pos1_v, pos2_v)
    wid = lax.axis_index("s") * 2 + lax.axis_index("c")
    base = wid * ROWS_PER_WORKER
    lanes = lax.iota(jnp.int32, 16)

    pltpu.sync_copy(idx_hbm.at[pl.ds(base, ROWS_PER_WORKER)], idx_v)
    tidx_v[pl.ds(0, 16)] = lanes // 8
    tidx_v[pl.ds(16, 16)] = (16 + lanes) // 8
    for t in range(NLISTS):
        pltpu.make_async_copy(
            table_hbm.at[tidx_v.at[pl.ds(t * 8, 8)]],
            rep_v.at[pl.ds(t * B, 8)],
            gsem,
        ).start()

    def comp(i, carry):
        idx16 = idx_v[pl.ds(i * 16, 16)]
        pos16 = base + i * 16 + lanes
        c0, c1 = carry
        m0 = idx16 == 0
        m1 = idx16 == 1
        s0 = c0 + plsc.cumsum(m0.astype(jnp.int32))
        s1 = c1 + plsc.cumsum(m1.astype(jnp.int32))
        s2 = i * 16 + lanes + 1 - s0 - s1
        for t, (m, s) in enumerate(((m0, s0), (m1, s1), (None, s2))):
            slots = s - 1
            plsc.store_scatter(
                pos_refs[t], [slots // B, slots % B], pos16,
                mask=(idx16 >= 2) if m is None else m,
            )
        return (
            c0 + plsc.all_reduce_population_count(m0),
            c1 + plsc.all_reduce_population_count(m1),
        )

    zero16 = jnp.zeros((16,), jnp.int32)
    counts_vec = lax.fori_loop(
        0, ROWS_PER_WORKER // 16, comp, (zero16, zero16), unroll=4
    )
    counts = (
        counts_vec[0][0],
        counts_vec[1][0],
        ROWS_PER_WORKER - counts_vec[0][0] - counts_vec[1][0],
    )

    nbatches = []
    for t in range(NLISTS):
        ct = counts[t]
        first16 = pos_refs[t][0, pl.ds(0, 16)]
        pad16 = jnp.zeros((16,), jnp.int32) + jnp.sum(
            jnp.where(lanes == 0, first16, 0)
        )
        for half in range(B // 16):
            slots = ct + half * 16 + lanes
            plsc.store_scatter(
                pos_refs[t], [slots // B, slots % B], pad16
            )
        nbatches.append((ct + B - 1) // B)

    for t in range(NLISTS):
        pltpu.make_async_copy(
            table_hbm.at[tidx_v.at[pl.ds(t * 8, 8)]],
            rep_v.at[pl.ds(t * B, 8)],
            gsem,
        ).wait()

    @pl.loop(8, B)
    def _(k):
        for t in range(NLISTS):
            for j in range(HIDDEN // 16):
                sl = pl.ds(j * 16, 16)
                rep_v[t * B + k, sl] = rep_v[t * B + (k % 8), sl]

    for t in range(NLISTS):
        def fire(b, carry, t=t):
            pltpu.make_async_copy(
                rep_v.at[pl.ds(t * B, B)],
                out_hbm.at[pos_refs[t].at[b]],
                sem,
            ).start()
            return carry

        lax.fori_loop(0, nbatches[t], fire, 0)

    total = nbatches[0] + nbatches[1] + nbatches[2]

    def drain(b, carry):
        pltpu.make_async_copy(
            rep_v.at[pl.ds(0, B)], out_hbm.at[pos0_v.at[0]], sem
        ).wait()
        return carry

    lax.fori_loop(0, total, drain, 0)


def kernel(indices, weight):
    idx = indices.reshape(-1).astype(jnp.int32)
    out = _scatter_kernel(idx, weight)
    return out.reshape(indices.shape + (weight.shape[1],))


# speedup vs baseline: 1.0354x; 1.0354x over previous
import functools

import jax
import jax.numpy as jnp
from jax import lax
from jax.experimental import pallas as pl
from jax.experimental.pallas import tpu as pltpu
from jax.experimental.pallas import tpu_sc as plsc

HIDDEN = 1024
TOTAL_ROWS = 4 * 8192
NUM_WORKERS = 32
ROWS_PER_WORKER = TOTAL_ROWS // NUM_WORKERS
NLISTS = 3
B = 32
NB_MAX = ROWS_PER_WORKER // B + 1

_mesh = plsc.VectorSubcoreMesh(core_axis_name="c", subcore_axis_name="s")


@functools.partial(
    pl.kernel,
    mesh=_mesh,
    out_type=(
        jax.ShapeDtypeStruct((TOTAL_ROWS, HIDDEN), jnp.float32),
        jax.ShapeDtypeStruct((NUM_WORKERS * 8, HIDDEN), jnp.float32),
    ),
    compiler_params=pltpu.CompilerParams(needs_layout_passes=False),
    scratch_types=[
        pltpu.VMEM((ROWS_PER_WORKER,), jnp.int32),
        pltpu.VMEM((NB_MAX, B), jnp.int32),
        pltpu.VMEM((NB_MAX, B), jnp.int32),
        pltpu.VMEM((NB_MAX, B), jnp.int32),
        pltpu.VMEM((96,), jnp.int32),
        pltpu.VMEM((8, HIDDEN), jnp.float32),
        pltpu.VMEM((NLISTS * B, HIDDEN), jnp.float32),
        pltpu.SemaphoreType.DMA,
        pltpu.SemaphoreType.DMA,
    ],
)
def _scatter_kernel(idx_hbm, table_hbm, out_hbm, wtab_hbm, idx_v, pos0_v,
                    pos1_v, pos2_v, tidx_v, tbuf_v, rep_v, gsem, sem):
    pos_refs = (pos0_v, pos1_v, pos2_v)
    wid = lax.axis_index("s") * 2 + lax.axis_index("c")
    base = wid * ROWS_PER_WORKER
    lanes = lax.iota(jnp.int32, 16)

    pltpu.sync_copy(table_hbm, tbuf_v.at[pl.ds(0, NLISTS)])
    pltpu.sync_copy(tbuf_v, wtab_hbm.at[pl.ds(wid * 8, 8)])
    pltpu.sync_copy(idx_hbm.at[pl.ds(base, ROWS_PER_WORKER)], idx_v)
    for j in range(6):
        tidx_v[pl.ds(j * 16, 16)] = wid * 8 + (j * 16 + lanes) // B
    for t in range(NLISTS):
        pltpu.make_async_copy(
            wtab_hbm.at[tidx_v.at[pl.ds(t * B, B)]],
            rep_v.at[pl.ds(t * B, B)],
            gsem,
        ).start()

    def comp(i, carry):
        idx16 = idx_v[pl.ds(i * 16, 16)]
        pos16 = base + i * 16 + lanes
        c0, c1 = carry
        m0 = idx16 == 0
        m1 = idx16 == 1
        s0 = c0 + plsc.cumsum(m0.astype(jnp.int32))
        s1 = c1 + plsc.cumsum(m1.astype(jnp.int32))
        s2 = i * 16 + lanes + 1 - s0 - s1
        for t, (m, s) in enumerate(((m0, s0), (m1, s1), (None, s2))):
            slots = s - 1
            plsc.store_scatter(
                pos_refs[t], [slots // B, slots % B], pos16,
                mask=(idx16 >= 2) if m is None else m,
            )
        return (
            c0 + plsc.all_reduce_population_count(m0),
            c1 + plsc.all_reduce_population_count(m1),
        )

    zero16 = jnp.zeros((16,), jnp.int32)
    counts_vec = lax.fori_loop(
        0, ROWS_PER_WORKER // 16, comp, (zero16, zero16), unroll=4
    )
    counts = (
        counts_vec[0][0],
        counts_vec[1][0],
        ROWS_PER_WORKER - counts_vec[0][0] - counts_vec[1][0],
    )

    nbatches = []
    for t in range(NLISTS):
        ct = counts[t]
        first16 = pos_refs[t][0, pl.ds(0, 16)]
        pad16 = jnp.zeros((16,), jnp.int32) + jnp.sum(
            jnp.where(lanes == 0, first16, 0)
        )
        for half in range(B // 16):
            slots = ct + half * 16 + lanes
            plsc.store_scatter(
                pos_refs[t], [slots // B, slots % B], pad16
            )
        nbatches.append((ct + B - 1) // B)


    for t in range(NLISTS):
        pltpu.make_async_copy(
            wtab_hbm.at[tidx_v.at[pl.ds(t * B, B)]],
            rep_v.at[pl.ds(t * B, B)],
            gsem,
        ).wait()

        def fire(b, carry, t=t):
            pltpu.make_async_copy(
                rep_v.at[pl.ds(t * B, B)],
                out_hbm.at[pos_refs[t].at[b]],
                sem,
            ).start()
            return carry

        lax.fori_loop(0, nbatches[t], fire, 0)

    total = nbatches[0] + nbatches[1] + nbatches[2]

    def drain(b, carry):
        pltpu.make_async_copy(
            rep_v.at[pl.ds(0, B)], out_hbm.at[pos0_v.at[0]], sem
        ).wait()
        return carry

    lax.fori_loop(0, total, drain, 0)


def kernel(indices, weight):
    idx = indices.reshape(-1).astype(jnp.int32)
    out, _ = _scatter_kernel(idx, weight)
    return out.reshape(indices.shape + (weight.shape[1],))


# speedup vs baseline: 1.2682x; 1.2249x over previous
import functools

import jax
import jax.numpy as jnp
from jax import lax
from jax.experimental import pallas as pl
from jax.experimental.pallas import tpu as pltpu
from jax.experimental.pallas import tpu_sc as plsc

HIDDEN = 1024
TOTAL_ROWS = 4 * 8192
NUM_WORKERS = 32
ROWS_PER_WORKER = TOTAL_ROWS // NUM_WORKERS
NLISTS = 3
B = 32
NB_MAX = ROWS_PER_WORKER // B + 1

_mesh = plsc.VectorSubcoreMesh(core_axis_name="c", subcore_axis_name="s")


@functools.partial(
    pl.kernel,
    mesh=_mesh,
    out_type=jax.ShapeDtypeStruct((TOTAL_ROWS, HIDDEN), jnp.float32),
    compiler_params=pltpu.CompilerParams(needs_layout_passes=False),
    scratch_types=[
        pltpu.VMEM((ROWS_PER_WORKER,), jnp.int32),
        pltpu.VMEM((NB_MAX, B), jnp.int32),
        pltpu.VMEM((NB_MAX, B), jnp.int32),
        pltpu.VMEM((NB_MAX, B), jnp.int32),
        pltpu.VMEM((NLISTS, HIDDEN), jnp.float32),
        pltpu.VMEM((NLISTS * B, HIDDEN), jnp.float32),
        pltpu.SemaphoreType.DMA,
    ],
)
def _scatter_kernel(idx_hbm, table_hbm, out_hbm, idx_v, pos0_v,
                    pos1_v, pos2_v, tbuf_v, rep_v, sem):
    pos_refs = (pos0_v, pos1_v, pos2_v)
    wid = lax.axis_index("s") * 2 + lax.axis_index("c")
    base = wid * ROWS_PER_WORKER
    lanes = lax.iota(jnp.int32, 16)

    pltpu.sync_copy(table_hbm, tbuf_v)
    pltpu.sync_copy(idx_hbm.at[pl.ds(base, ROWS_PER_WORKER)], idx_v)

    @pl.loop(0, HIDDEN // 16)
    def _(j):
        sl = pl.ds(j * 16, 16)
        for t in range(NLISTS):
            w = tbuf_v[t, sl]
            for k in range(B):
                rep_v[t * B + k, sl] = w

    def comp(i, carry):
        idx16 = idx_v[pl.ds(i * 16, 16)]
        pos16 = base + i * 16 + lanes
        c0, c1 = carry
        m0 = idx16 == 0
        m1 = idx16 == 1
        s0 = c0 + plsc.cumsum(m0.astype(jnp.int32))
        s1 = c1 + plsc.cumsum(m1.astype(jnp.int32))
        s2 = i * 16 + lanes + 1 - s0 - s1
        for t, (m, s) in enumerate(((m0, s0), (m1, s1), (None, s2))):
            slots = s - 1
            plsc.store_scatter(
                pos_refs[t], [slots // B, slots % B], pos16,
                mask=(idx16 >= 2) if m is None else m,
            )
        return (
            c0 + plsc.all_reduce_population_count(m0),
            c1 + plsc.all_reduce_population_count(m1),
        )

    zero16 = jnp.zeros((16,), jnp.int32)
    counts_vec = lax.fori_loop(
        0, ROWS_PER_WORKER // 16, comp, (zero16, zero16), unroll=4
    )
    counts = (
        counts_vec[0][0],
        counts_vec[1][0],
        ROWS_PER_WORKER - counts_vec[0][0] - counts_vec[1][0],
    )

    nbatches = []
    for t in range(NLISTS):
        ct = counts[t]
        first16 = pos_refs[t][0, pl.ds(0, 16)]
        pad16 = jnp.zeros((16,), jnp.int32) + jnp.sum(
            jnp.where(lanes == 0, first16, 0)
        )
        for half in range(B // 16):
            slots = ct + half * 16 + lanes
            plsc.store_scatter(
                pos_refs[t], [slots // B, slots % B], pad16
            )
        nbatches.append((ct + B - 1) // B)


    for t in range(NLISTS):
        def fire(b, carry, t=t):
            pltpu.make_async_copy(
                rep_v.at[pl.ds(t * B, B)],
                out_hbm.at[pos_refs[t].at[b]],
                sem,
            ).start()
            return carry

        lax.fori_loop(0, nbatches[t], fire, 0)

    total = nbatches[0] + nbatches[1] + nbatches[2]

    def drain(b, carry):
        pltpu.make_async_copy(
            rep_v.at[pl.ds(0, B)], out_hbm.at[pos0_v.at[0]], sem
        ).wait()
        return carry

    lax.fori_loop(0, total, drain, 0)


def kernel(indices, weight):
    idx = indices.reshape(-1).astype(jnp.int32)
    out = _scatter_kernel(idx, weight)
    return out.reshape(indices.shape + (weight.shape[1],))
